# baseline (speedup 1.0000x reference)
import functools

import numpy as np

import jax
import jax.numpy as jnp
from jax import lax
from jax.experimental import pallas as pl
from jax.experimental.pallas import tpu as pltpu
from jax.experimental.pallas import tpu_sc as plsc

N_NODES = 10000
N_EDGES = 320000
D_SCALAR = 128
D_EDGE = 16
D_HIDDEN = 128

_PERM = np.empty(D_HIDDEN, dtype=np.int32)
for _g in range(4):
    for _p in range(32):
        _PERM[32 * _g + _p] = 32 * _g + 16 * (_p % 2) + _p // 2
_PERM_INV = np.argsort(_PERM).astype(np.int32)

_A_BLK = 1000


def _tables_body(ns_ref, w_ref, out_ref):
    out_ref[...] = jnp.dot(ns_ref[...], w_ref[...],
                           preferred_element_type=jnp.float32)


def _node_tables(node_scalars, W1):
    return pl.pallas_call(
        _tables_body,
        grid=(2, N_NODES // _A_BLK),
        in_specs=[
            pl.BlockSpec((_A_BLK, D_SCALAR), lambda t, i: (i, 0)),
            pl.BlockSpec((D_SCALAR, D_HIDDEN), lambda t, i: (t, 0)),
        ],
        out_specs=pl.BlockSpec((_A_BLK, D_HIDDEN),
                               lambda t, i: (t * (N_NODES // _A_BLK) + i, 0)),
        out_shape=jax.ShapeDtypeStruct((2 * N_NODES, D_HIDDEN), jnp.float32),
    )(node_scalars, W1)


_NW = 32
_SPLITS = (192000, 89600, 38400)
_C = 400


_GATHER_ADD_CACHE = {}


def _gather_add_build(ebase, e_chunk):
    key = (ebase, e_chunk)
    if key in _GATHER_ADD_CACHE:
        return _GATHER_ADD_CACHE[key]
    _EP = e_chunk // _NW
    _NCHUNK = _EP // _C
    mesh = plsc.VectorSubcoreMesh(core_axis_name="c", subcore_axis_name="s")

    @functools.partial(
        pl.kernel,
        out_type=jax.ShapeDtypeStruct((e_chunk, D_HIDDEN), jnp.float32),
        mesh=mesh,
        scratch_types=[
            pltpu.VMEM((_EP,), jnp.int32),
            pltpu.VMEM((_EP,), jnp.int32),
            pltpu.VMEM((_C, D_HIDDEN), jnp.float32),
            pltpu.VMEM((_C, D_HIDDEN), jnp.float32),
            pltpu.SemaphoreType.DMA,
            pltpu.SemaphoreType.DMA,
            pltpu.SemaphoreType.DMA,
            pltpu.SemaphoreType.DMA,
        ],
    )
    def gather_add(table_hbm, src_hbm, dst_hbm, out_hbm,
                   idx_s, idx_d, buf0, buf1, gs0, gs1, ws0, ws1):
        wid = lax.axis_index("s") * 2 + lax.axis_index("c")
        base = pl.multiple_of(wid * _EP, 8)
        bufs = (buf0, buf1)
        gsems = (gs0, gs1)
        wsems = (ws0, ws1)

        pltpu.sync_copy(src_hbm.at[pl.ds(ebase + base, _EP)], idx_s)
        pltpu.sync_copy(dst_hbm.at[pl.ds(ebase + base, _EP)], idx_d)

        def off_row(i, c):
            sl = pl.ds(i * 16, 16)
            idx_d[sl] = idx_d[sl] + N_NODES
            return c

        lax.fori_loop(0, _EP // 16, off_row, 0)

        def g1(ci):
            s = ci % 2
            return pltpu.async_copy(
                table_hbm.at[idx_s.at[pl.ds(ci * _C, _C)]], bufs[s], gsems[s])

        def g2(ci):
            s = ci % 2
            return pltpu.async_copy(
                table_hbm.at[idx_d.at[pl.ds(ci * _C, _C)]], bufs[s], gsems[s],
                add=True)

        def wb(ci):
            s = ci % 2
            return pltpu.async_copy(
                bufs[s], out_hbm.at[pl.ds(base + ci * _C, _C)], wsems[s])

        wbd = [None] * _NCHUNK
        d = g1(0)
        for ci in range(_NCHUNK):
            d.wait()
            dg2 = g2(ci)
            if ci >= 1:
                wbd[ci - 1].wait()
            if ci + 1 < _NCHUNK:
                d = g1(ci + 1)
            dg2.wait()
            wbd[ci] = wb(ci)
        wbd[_NCHUNK - 1].wait()

    _GATHER_ADD_CACHE[key] = gather_add
    return gather_add


_E_BLK = 3200


def _tail_body(g_ref, eft_ref, w1e_ref, b1_ref, w2_ref, b2t_ref, out_ref):
    x = (g_ref[...]
         + lax.dot_general(eft_ref[...], w1e_ref[...],
                           (((0,), (0,)), ((), ())),
                           preferred_element_type=jnp.float32)
         + b1_ref[...])
    h = x * jax.nn.sigmoid(x)
    out_ref[...] = (lax.dot_general(w2_ref[...], h,
                                    (((0,), (1,)), ((), ())),
                                    preferred_element_type=jnp.float32)
                    + b2t_ref[...])


def _tail_body_aliased(buf_ref, g_ref, eft_ref, w1e_ref, b1_ref, w2_ref,
                       b2t_ref, out_ref):
    del buf_ref
    _tail_body(g_ref, eft_ref, w1e_ref, b1_ref, w2_ref, b2t_ref, out_ref)


def _mlp_tail(g, ef_t, w1e, b1, w2, b2t, buf, ebase):
    n = g.shape[0]
    nblk = n // _E_BLK
    blk0 = ebase // _E_BLK
    col_spec = pl.BlockSpec((D_EDGE, _E_BLK), lambda i: (0, blk0 + i))
    data_specs = [
        pl.BlockSpec((_E_BLK, D_HIDDEN), lambda i: (i, 0)),
        col_spec,
        pl.BlockSpec((D_EDGE, D_HIDDEN), lambda i: (0, 0)),
        pl.BlockSpec((1, D_HIDDEN), lambda i: (0, 0)),
        pl.BlockSpec((D_HIDDEN, D_EDGE), lambda i: (0, 0)),
        pl.BlockSpec((D_EDGE, 1), lambda i: (0, 0)),
    ]
    out_shape = jax.ShapeDtypeStruct((D_EDGE, N_EDGES), jnp.float32)
    if buf is None:
        return pl.pallas_call(
            _tail_body,
            grid=(nblk,),
            in_specs=data_specs,
            out_specs=col_spec,
            out_shape=out_shape,
        )(g, ef_t, w1e, b1, w2, b2t)
    return pl.pallas_call(
        _tail_body_aliased,
        grid=(nblk,),
        in_specs=[pl.BlockSpec(memory_space=pl.ANY)] + data_specs,
        out_specs=col_spec,
        out_shape=out_shape,
        input_output_aliases={0: 0},
    )(buf, g, ef_t, w1e, b1, w2, b2t)


def kernel(node_scalars, edge_index, edge_feats, W1, b1, W2, b2):
    ei = edge_index.astype(jnp.int32)
    src = ei[0]
    dst = ei[1]
    ef_t = edge_feats.T
    w1e = W1[2 * D_SCALAR:]
    b1r = b1.reshape(1, D_HIDDEN)
    b2t = b2.reshape(D_EDGE, 1)
    table = _node_tables(node_scalars, W1)
    buf = None
    ebase = 0
    for sz in _SPLITS:
        g_k = _gather_add_build(ebase, sz)(table, src, dst)
        buf = _mlp_tail(g_k, ef_t, w1e, b1r, W2, b2t, buf, ebase)
        ebase += sz
    return buf.T


# speedup vs baseline: 1.0151x; 1.0151x over previous
import functools

import numpy as np

import jax
import jax.numpy as jnp
from jax import lax
from jax.experimental import pallas as pl
from jax.experimental.pallas import tpu as pltpu
from jax.experimental.pallas import tpu_sc as plsc

N_NODES = 10000
N_EDGES = 320000
D_SCALAR = 128
D_EDGE = 16
D_HIDDEN = 128

_PERM = np.empty(D_HIDDEN, dtype=np.int32)
for _g in range(4):
    for _p in range(32):
        _PERM[32 * _g + _p] = 32 * _g + 16 * (_p % 2) + _p // 2
_PERM_INV = np.argsort(_PERM).astype(np.int32)

_A_BLK = 1000


def _tables_body(ns_ref, w_ref, out_ref):
    out_ref[...] = jnp.dot(ns_ref[...].astype(jnp.bfloat16),
                           w_ref[...].astype(jnp.bfloat16),
                           preferred_element_type=jnp.float32)


def _node_tables(node_scalars, W1):
    return pl.pallas_call(
        _tables_body,
        grid=(2, N_NODES // _A_BLK),
        in_specs=[
            pl.BlockSpec((_A_BLK, D_SCALAR), lambda t, i: (i, 0)),
            pl.BlockSpec((D_SCALAR, D_HIDDEN), lambda t, i: (t, 0)),
        ],
        out_specs=pl.BlockSpec((_A_BLK, D_HIDDEN),
                               lambda t, i: (t * (N_NODES // _A_BLK) + i, 0)),
        out_shape=jax.ShapeDtypeStruct((2 * N_NODES, D_HIDDEN), jnp.float32),
    )(node_scalars, W1)


_NW = 32
_SPLITS = (N_EDGES,)
_C = 400


_GATHER_ADD_CACHE = {}


def _gather_add_build(ebase, e_chunk):
    key = (ebase, e_chunk)
    if key in _GATHER_ADD_CACHE:
        return _GATHER_ADD_CACHE[key]
    _EP = e_chunk // _NW
    _NCHUNK = _EP // _C
    mesh = plsc.VectorSubcoreMesh(core_axis_name="c", subcore_axis_name="s")

    @functools.partial(
        pl.kernel,
        out_type=jax.ShapeDtypeStruct((e_chunk, D_HIDDEN), jnp.float32),
        mesh=mesh,
        scratch_types=[
            pltpu.VMEM((_EP,), jnp.int32),
            pltpu.VMEM((_EP,), jnp.int32),
            pltpu.VMEM((_C, D_HIDDEN), jnp.float32),
            pltpu.VMEM((_C, D_HIDDEN), jnp.float32),
            pltpu.SemaphoreType.DMA,
            pltpu.SemaphoreType.DMA,
            pltpu.SemaphoreType.DMA,
            pltpu.SemaphoreType.DMA,
        ],
    )
    def gather_add(table_hbm, src_hbm, dst_hbm, out_hbm,
                   idx_s, idx_d, buf0, buf1, gs0, gs1, ws0, ws1):
        wid = lax.axis_index("s") * 2 + lax.axis_index("c")
        base = pl.multiple_of(wid * _EP, 8)
        bufs = (buf0, buf1)
        gsems = (gs0, gs1)
        wsems = (ws0, ws1)

        pltpu.sync_copy(src_hbm.at[pl.ds(ebase + base, _EP)], idx_s)
        pltpu.sync_copy(dst_hbm.at[pl.ds(ebase + base, _EP)], idx_d)

        def off_row(i, c):
            sl = pl.ds(i * 16, 16)
            idx_d[sl] = idx_d[sl] + N_NODES
            return c

        lax.fori_loop(0, _EP // 16, off_row, 0)

        def g1(ci):
            s = ci % 2
            return pltpu.async_copy(
                table_hbm.at[idx_s.at[pl.ds(ci * _C, _C)]], bufs[s], gsems[s])

        def g2(ci):
            s = ci % 2
            return pltpu.async_copy(
                table_hbm.at[idx_d.at[pl.ds(ci * _C, _C)]], bufs[s], gsems[s],
                add=True)

        def wb(ci):
            s = ci % 2
            return pltpu.async_copy(
                bufs[s], out_hbm.at[pl.ds(base + ci * _C, _C)], wsems[s])

        wbd = [None] * _NCHUNK
        d = g1(0)
        for ci in range(_NCHUNK):
            d.wait()
            dg2 = g2(ci)
            if ci >= 1:
                wbd[ci - 1].wait()
            if ci + 1 < _NCHUNK:
                d = g1(ci + 1)
            dg2.wait()
            wbd[ci] = wb(ci)
        wbd[_NCHUNK - 1].wait()

    _GATHER_ADD_CACHE[key] = gather_add
    return gather_add


_E_BLK = 6400


def _tail_body(g_ref, eft_ref, w1e_ref, b1_ref, w2_ref, b2t_ref, out_ref):
    x = (g_ref[...]
         + lax.dot_general(eft_ref[...], w1e_ref[...],
                           (((0,), (0,)), ((), ())),
                           preferred_element_type=jnp.float32)
         + b1_ref[...])
    h = x * jax.nn.sigmoid(x)
    out_ref[...] = (lax.dot_general(w2_ref[...], h,
                                    (((0,), (1,)), ((), ())),
                                    preferred_element_type=jnp.float32)
                    + b2t_ref[...])


def _tail_body_aliased(buf_ref, g_ref, eft_ref, w1e_ref, b1_ref, w2_ref,
                       b2t_ref, out_ref):
    del buf_ref
    _tail_body(g_ref, eft_ref, w1e_ref, b1_ref, w2_ref, b2t_ref, out_ref)


def _mlp_tail(g, ef_t, w1e, b1, w2, b2t, buf, ebase):
    n = g.shape[0]
    nblk = n // _E_BLK
    blk0 = ebase // _E_BLK
    col_spec = pl.BlockSpec((D_EDGE, _E_BLK), lambda i: (0, blk0 + i))
    data_specs = [
        pl.BlockSpec((_E_BLK, D_HIDDEN), lambda i: (i, 0)),
        col_spec,
        pl.BlockSpec((D_EDGE, D_HIDDEN), lambda i: (0, 0)),
        pl.BlockSpec((1, D_HIDDEN), lambda i: (0, 0)),
        pl.BlockSpec((D_HIDDEN, D_EDGE), lambda i: (0, 0)),
        pl.BlockSpec((D_EDGE, 1), lambda i: (0, 0)),
    ]
    out_shape = jax.ShapeDtypeStruct((D_EDGE, N_EDGES), jnp.float32)
    if buf is None:
        return pl.pallas_call(
            _tail_body,
            grid=(nblk,),
            in_specs=data_specs,
            out_specs=col_spec,
            out_shape=out_shape,
        )(g, ef_t, w1e, b1, w2, b2t)
    return pl.pallas_call(
        _tail_body_aliased,
        grid=(nblk,),
        in_specs=[pl.BlockSpec(memory_space=pl.ANY)] + data_specs,
        out_specs=col_spec,
        out_shape=out_shape,
        input_output_aliases={0: 0},
    )(buf, g, ef_t, w1e, b1, w2, b2t)


def kernel(node_scalars, edge_index, edge_feats, W1, b1, W2, b2):
    ei = edge_index.astype(jnp.int32)
    src = ei[0]
    dst = ei[1]
    ef_t = edge_feats.T
    w1e = W1[2 * D_SCALAR:]
    b1r = b1.reshape(1, D_HIDDEN)
    b2t = b2.reshape(D_EDGE, 1)
    table = _node_tables(node_scalars, W1)
    buf = None
    ebase = 0
    for sz in _SPLITS:
        g_k = _gather_add_build(ebase, sz)(table, src, dst)
        buf = _mlp_tail(g_k, ef_t, w1e, b1r, W2, b2t, buf, ebase)
        ebase += sz
    return buf.T


# speedup vs baseline: 1.0377x; 1.0223x over previous
import functools

import numpy as np

import jax
import jax.numpy as jnp
from jax import lax
from jax.experimental import pallas as pl
from jax.experimental.pallas import tpu as pltpu
from jax.experimental.pallas import tpu_sc as plsc

N_NODES = 10000
N_EDGES = 320000
D_SCALAR = 128
D_EDGE = 16
D_HIDDEN = 128

_PERM = np.empty(D_HIDDEN, dtype=np.int32)
for _g in range(4):
    for _p in range(32):
        _PERM[32 * _g + _p] = 32 * _g + 16 * (_p % 2) + _p // 2
_PERM_INV = np.argsort(_PERM).astype(np.int32)

_A_BLK = 1000


def _tables_body(ns_ref, w_ref, out_ref):
    out_ref[...] = jnp.dot(ns_ref[...].astype(jnp.bfloat16),
                           w_ref[...].astype(jnp.bfloat16),
                           preferred_element_type=jnp.float32)


def _node_tables(node_scalars, W1):
    return pl.pallas_call(
        _tables_body,
        grid=(2, N_NODES // _A_BLK),
        in_specs=[
            pl.BlockSpec((_A_BLK, D_SCALAR), lambda t, i: (i, 0)),
            pl.BlockSpec((D_SCALAR, D_HIDDEN), lambda t, i: (t, 0)),
        ],
        out_specs=pl.BlockSpec((_A_BLK, D_HIDDEN),
                               lambda t, i: (t * (N_NODES // _A_BLK) + i, 0)),
        out_shape=jax.ShapeDtypeStruct((2 * N_NODES, D_HIDDEN), jnp.float32),
    )(node_scalars, W1)


_NW = 32
_SPLITS = (256000, 64000)
_C = 400


_GATHER_ADD_CACHE = {}


def _gather_add_build(ebase, e_chunk):
    key = (ebase, e_chunk)
    if key in _GATHER_ADD_CACHE:
        return _GATHER_ADD_CACHE[key]
    _EP = e_chunk // _NW
    _NCHUNK = _EP // _C
    mesh = plsc.VectorSubcoreMesh(core_axis_name="c", subcore_axis_name="s")

    @functools.partial(
        pl.kernel,
        out_type=jax.ShapeDtypeStruct((e_chunk, D_HIDDEN), jnp.float32),
        mesh=mesh,
        scratch_types=[
            pltpu.VMEM((_EP,), jnp.int32),
            pltpu.VMEM((_EP,), jnp.int32),
            pltpu.VMEM((_C, D_HIDDEN), jnp.float32),
            pltpu.VMEM((_C, D_HIDDEN), jnp.float32),
            pltpu.SemaphoreType.DMA,
            pltpu.SemaphoreType.DMA,
            pltpu.SemaphoreType.DMA,
            pltpu.SemaphoreType.DMA,
        ],
    )
    def gather_add(table_hbm, src_hbm, dst_hbm, out_hbm,
                   idx_s, idx_d, buf0, buf1, gs0, gs1, ws0, ws1):
        wid = lax.axis_index("s") * 2 + lax.axis_index("c")
        base = pl.multiple_of(wid * _EP, 8)
        bufs = (buf0, buf1)
        gsems = (gs0, gs1)
        wsems = (ws0, ws1)

        pltpu.sync_copy(src_hbm.at[pl.ds(ebase + base, _EP)], idx_s)
        pltpu.sync_copy(dst_hbm.at[pl.ds(ebase + base, _EP)], idx_d)

        def off_row(i, c):
            sl = pl.ds(i * 16, 16)
            idx_d[sl] = idx_d[sl] + N_NODES
            return c

        lax.fori_loop(0, _EP // 16, off_row, 0)

        def g1(ci):
            s = ci % 2
            return pltpu.async_copy(
                table_hbm.at[idx_s.at[pl.ds(ci * _C, _C)]], bufs[s], gsems[s])

        def g2(ci):
            s = ci % 2
            return pltpu.async_copy(
                table_hbm.at[idx_d.at[pl.ds(ci * _C, _C)]], bufs[s], gsems[s],
                add=True)

        def wb(ci):
            s = ci % 2
            return pltpu.async_copy(
                bufs[s], out_hbm.at[pl.ds(base + ci * _C, _C)], wsems[s])

        wbd = [None] * _NCHUNK
        d = g1(0)
        for ci in range(_NCHUNK):
            d.wait()
            dg2 = g2(ci)
            if ci >= 1:
                wbd[ci - 1].wait()
            if ci + 1 < _NCHUNK:
                d = g1(ci + 1)
            dg2.wait()
            wbd[ci] = wb(ci)
        wbd[_NCHUNK - 1].wait()

    _GATHER_ADD_CACHE[key] = gather_add
    return gather_add


_E_BLK = 6400


def _tail_body(g_ref, eft_ref, w1e_ref, b1_ref, w2_ref, b2t_ref, out_ref):
    x = (g_ref[...]
         + lax.dot_general(eft_ref[...], w1e_ref[...],
                           (((0,), (0,)), ((), ())),
                           preferred_element_type=jnp.float32)
         + b1_ref[...])
    h = x * jax.nn.sigmoid(x)
    out_ref[...] = (lax.dot_general(w2_ref[...], h,
                                    (((0,), (1,)), ((), ())),
                                    preferred_element_type=jnp.float32)
                    + b2t_ref[...])


def _tail_body_aliased(buf_ref, g_ref, eft_ref, w1e_ref, b1_ref, w2_ref,
                       b2t_ref, out_ref):
    del buf_ref
    _tail_body(g_ref, eft_ref, w1e_ref, b1_ref, w2_ref, b2t_ref, out_ref)


def _mlp_tail(g, ef_t, w1e, b1, w2, b2t, buf, ebase):
    n = g.shape[0]
    nblk = n // _E_BLK
    blk0 = ebase // _E_BLK
    col_spec = pl.BlockSpec((D_EDGE, _E_BLK), lambda i: (0, blk0 + i))
    data_specs = [
        pl.BlockSpec((_E_BLK, D_HIDDEN), lambda i: (i, 0)),
        col_spec,
        pl.BlockSpec((D_EDGE, D_HIDDEN), lambda i: (0, 0)),
        pl.BlockSpec((1, D_HIDDEN), lambda i: (0, 0)),
        pl.BlockSpec((D_HIDDEN, D_EDGE), lambda i: (0, 0)),
        pl.BlockSpec((D_EDGE, 1), lambda i: (0, 0)),
    ]
    out_shape = jax.ShapeDtypeStruct((D_EDGE, N_EDGES), jnp.float32)
    if buf is None:
        return pl.pallas_call(
            _tail_body,
            grid=(nblk,),
            in_specs=data_specs,
            out_specs=col_spec,
            out_shape=out_shape,
        )(g, ef_t, w1e, b1, w2, b2t)
    return pl.pallas_call(
        _tail_body_aliased,
        grid=(nblk,),
        in_specs=[pl.BlockSpec(memory_space=pl.ANY)] + data_specs,
        out_specs=col_spec,
        out_shape=out_shape,
        input_output_aliases={0: 0},
    )(buf, g, ef_t, w1e, b1, w2, b2t)


def kernel(node_scalars, edge_index, edge_feats, W1, b1, W2, b2):
    ei = edge_index.astype(jnp.int32)
    src = ei[0]
    dst = ei[1]
    ef_t = edge_feats.T
    w1e = W1[2 * D_SCALAR:]
    b1r = b1.reshape(1, D_HIDDEN)
    b2t = b2.reshape(D_EDGE, 1)
    table = _node_tables(node_scalars, W1)
    buf = None
    ebase = 0
    for sz in _SPLITS:
        g_k = _gather_add_build(ebase, sz)(table, src, dst)
        buf = _mlp_tail(g_k, ef_t, w1e, b1r, W2, b2t, buf, ebase)
        ebase += sz
    return buf.T


# speedup vs baseline: 1.0385x; 1.0007x over previous
import functools

import numpy as np

import jax
import jax.numpy as jnp
from jax import lax
from jax.experimental import pallas as pl
from jax.experimental.pallas import tpu as pltpu
from jax.experimental.pallas import tpu_sc as plsc

N_NODES = 10000
N_EDGES = 320000
D_SCALAR = 128
D_EDGE = 16
D_HIDDEN = 128

_PERM = np.empty(D_HIDDEN, dtype=np.int32)
for _g in range(4):
    for _p in range(32):
        _PERM[32 * _g + _p] = 32 * _g + 16 * (_p % 2) + _p // 2
_PERM_INV = np.argsort(_PERM).astype(np.int32)

_A_BLK = 1000


def _tables_body(ns_ref, w_ref, out_ref):
    out_ref[...] = jnp.dot(ns_ref[...].astype(jnp.bfloat16),
                           w_ref[...].astype(jnp.bfloat16),
                           preferred_element_type=jnp.float32)


def _node_tables(node_scalars, W1):
    return pl.pallas_call(
        _tables_body,
        grid=(2, N_NODES // _A_BLK),
        in_specs=[
            pl.BlockSpec((_A_BLK, D_SCALAR), lambda t, i: (i, 0)),
            pl.BlockSpec((D_SCALAR, D_HIDDEN), lambda t, i: (t, 0)),
        ],
        out_specs=pl.BlockSpec((_A_BLK, D_HIDDEN),
                               lambda t, i: (t * (N_NODES // _A_BLK) + i, 0)),
        out_shape=jax.ShapeDtypeStruct((2 * N_NODES, D_HIDDEN), jnp.float32),
    )(node_scalars, W1)


_NW = 32
_SPLITS = (256000, 38400, 25600)
_C = 400


_GATHER_ADD_CACHE = {}


def _gather_add_build(ebase, e_chunk):
    key = (ebase, e_chunk)
    if key in _GATHER_ADD_CACHE:
        return _GATHER_ADD_CACHE[key]
    _EP = e_chunk // _NW
    _NCHUNK = _EP // _C
    mesh = plsc.VectorSubcoreMesh(core_axis_name="c", subcore_axis_name="s")

    @functools.partial(
        pl.kernel,
        out_type=jax.ShapeDtypeStruct((e_chunk, D_HIDDEN), jnp.float32),
        mesh=mesh,
        scratch_types=[
            pltpu.VMEM((_EP,), jnp.int32),
            pltpu.VMEM((_EP,), jnp.int32),
            pltpu.VMEM((_C, D_HIDDEN), jnp.float32),
            pltpu.VMEM((_C, D_HIDDEN), jnp.float32),
            pltpu.SemaphoreType.DMA,
            pltpu.SemaphoreType.DMA,
            pltpu.SemaphoreType.DMA,
            pltpu.SemaphoreType.DMA,
        ],
    )
    def gather_add(table_hbm, src_hbm, dst_hbm, out_hbm,
                   idx_s, idx_d, buf0, buf1, gs0, gs1, ws0, ws1):
        wid = lax.axis_index("s") * 2 + lax.axis_index("c")
        base = pl.multiple_of(wid * _EP, 8)
        bufs = (buf0, buf1)
        gsems = (gs0, gs1)
        wsems = (ws0, ws1)

        pltpu.sync_copy(src_hbm.at[pl.ds(ebase + base, _EP)], idx_s)
        pltpu.sync_copy(dst_hbm.at[pl.ds(ebase + base, _EP)], idx_d)

        def off_row(i, c):
            sl = pl.ds(i * 16, 16)
            idx_d[sl] = idx_d[sl] + N_NODES
            return c

        lax.fori_loop(0, _EP // 16, off_row, 0)

        def g1(ci):
            s = ci % 2
            return pltpu.async_copy(
                table_hbm.at[idx_s.at[pl.ds(ci * _C, _C)]], bufs[s], gsems[s])

        def g2(ci):
            s = ci % 2
            return pltpu.async_copy(
                table_hbm.at[idx_d.at[pl.ds(ci * _C, _C)]], bufs[s], gsems[s],
                add=True)

        def wb(ci):
            s = ci % 2
            return pltpu.async_copy(
                bufs[s], out_hbm.at[pl.ds(base + ci * _C, _C)], wsems[s])

        wbd = [None] * _NCHUNK
        d = g1(0)
        for ci in range(_NCHUNK):
            d.wait()
            dg2 = g2(ci)
            if ci >= 1:
                wbd[ci - 1].wait()
            if ci + 1 < _NCHUNK:
                d = g1(ci + 1)
            dg2.wait()
            wbd[ci] = wb(ci)
        wbd[_NCHUNK - 1].wait()

    _GATHER_ADD_CACHE[key] = gather_add
    return gather_add


_E_BLK = 6400


def _tail_body(g_ref, eft_ref, w1e_ref, b1_ref, w2_ref, b2t_ref, out_ref):
    x = (g_ref[...]
         + lax.dot_general(eft_ref[...], w1e_ref[...],
                           (((0,), (0,)), ((), ())),
                           preferred_element_type=jnp.float32)
         + b1_ref[...])
    h = x * jax.nn.sigmoid(x)
    out_ref[...] = (lax.dot_general(w2_ref[...], h,
                                    (((0,), (1,)), ((), ())),
                                    preferred_element_type=jnp.float32)
                    + b2t_ref[...])


def _tail_body_aliased(buf_ref, g_ref, eft_ref, w1e_ref, b1_ref, w2_ref,
                       b2t_ref, out_ref):
    del buf_ref
    _tail_body(g_ref, eft_ref, w1e_ref, b1_ref, w2_ref, b2t_ref, out_ref)


def _mlp_tail(g, ef_t, w1e, b1, w2, b2t, buf, ebase):
    n = g.shape[0]
    nblk = n // _E_BLK
    blk0 = ebase // _E_BLK
    col_spec = pl.BlockSpec((D_EDGE, _E_BLK), lambda i: (0, blk0 + i))
    data_specs = [
        pl.BlockSpec((_E_BLK, D_HIDDEN), lambda i: (i, 0)),
        col_spec,
        pl.BlockSpec((D_EDGE, D_HIDDEN), lambda i: (0, 0)),
        pl.BlockSpec((1, D_HIDDEN), lambda i: (0, 0)),
        pl.BlockSpec((D_HIDDEN, D_EDGE), lambda i: (0, 0)),
        pl.BlockSpec((D_EDGE, 1), lambda i: (0, 0)),
    ]
    out_shape = jax.ShapeDtypeStruct((D_EDGE, N_EDGES), jnp.float32)
    if buf is None:
        return pl.pallas_call(
            _tail_body,
            grid=(nblk,),
            in_specs=data_specs,
            out_specs=col_spec,
            out_shape=out_shape,
        )(g, ef_t, w1e, b1, w2, b2t)
    return pl.pallas_call(
        _tail_body_aliased,
        grid=(nblk,),
        in_specs=[pl.BlockSpec(memory_space=pl.ANY)] + data_specs,
        out_specs=col_spec,
        out_shape=out_shape,
        input_output_aliases={0: 0},
    )(buf, g, ef_t, w1e, b1, w2, b2t)


def kernel(node_scalars, edge_index, edge_feats, W1, b1, W2, b2):
    ei = edge_index.astype(jnp.int32)
    src = ei[0]
    dst = ei[1]
    ef_t = edge_feats.T
    w1e = W1[2 * D_SCALAR:]
    b1r = b1.reshape(1, D_HIDDEN)
    b2t = b2.reshape(D_EDGE, 1)
    table = _node_tables(node_scalars, W1)
    buf = None
    ebase = 0
    for sz in _SPLITS:
        g_k = _gather_add_build(ebase, sz)(table, src, dst)
        buf = _mlp_tail(g_k, ef_t, w1e, b1r, W2, b2t, buf, ebase)
        ebase += sz
    return buf.T


# speedup vs baseline: 1.0386x; 1.0001x over previous
import functools

import jax
import jax.numpy as jnp
from jax import lax
from jax.experimental import pallas as pl
from jax.experimental.pallas import tpu as pltpu
from jax.experimental.pallas import tpu_sc as plsc

N_NODES = 10000
N_EDGES = 320000
D_SCALAR = 128
D_EDGE = 16
D_HIDDEN = 128

_A_BLK = 1000


def _tables_body(ns_ref, w_ref, out_ref):
    out_ref[...] = jnp.dot(ns_ref[...].astype(jnp.bfloat16),
                           w_ref[...].astype(jnp.bfloat16),
                           preferred_element_type=jnp.float32)


def _node_tables(node_scalars, W1):
    return pl.pallas_call(
        _tables_body,
        grid=(2, N_NODES // _A_BLK),
        in_specs=[
            pl.BlockSpec((_A_BLK, D_SCALAR), lambda t, i: (i, 0)),
            pl.BlockSpec((D_SCALAR, D_HIDDEN), lambda t, i: (t, 0)),
        ],
        out_specs=pl.BlockSpec((_A_BLK, D_HIDDEN),
                               lambda t, i: (t * (N_NODES // _A_BLK) + i, 0)),
        out_shape=jax.ShapeDtypeStruct((2 * N_NODES, D_HIDDEN), jnp.float32),
    )(node_scalars, W1)


_NW = 32
_SPLITS = (256000, 64000)
_C = 400


_GATHER_ADD_CACHE = {}


def _gather_add_build(ebase, e_chunk):
    key = (ebase, e_chunk)
    if key in _GATHER_ADD_CACHE:
        return _GATHER_ADD_CACHE[key]
    _EP = e_chunk // _NW
    _NCHUNK = _EP // _C
    mesh = plsc.VectorSubcoreMesh(core_axis_name="c", subcore_axis_name="s")

    @functools.partial(
        pl.kernel,
        out_type=jax.ShapeDtypeStruct((e_chunk, D_HIDDEN), jnp.float32),
        mesh=mesh,
        scratch_types=[
            pltpu.VMEM((_EP,), jnp.int32),
            pltpu.VMEM((_EP,), jnp.int32),
            pltpu.VMEM((_C, D_HIDDEN), jnp.float32),
            pltpu.VMEM((_C, D_HIDDEN), jnp.float32),
            pltpu.SemaphoreType.DMA,
            pltpu.SemaphoreType.DMA,
            pltpu.SemaphoreType.DMA,
            pltpu.SemaphoreType.DMA,
        ],
    )
    def gather_add(table_hbm, src_hbm, dst_hbm, out_hbm,
                   idx_s, idx_d, buf0, buf1, gs0, gs1, ws0, ws1):
        wid = lax.axis_index("s") * 2 + lax.axis_index("c")
        base = pl.multiple_of(wid * _EP, 8)
        bufs = (buf0, buf1)
        gsems = (gs0, gs1)
        wsems = (ws0, ws1)

        pltpu.sync_copy(src_hbm.at[pl.ds(ebase + base, _EP)], idx_s)
        pltpu.sync_copy(dst_hbm.at[pl.ds(ebase + base, _EP)], idx_d)

        def off_row(i, c):
            sl = pl.ds(i * 16, 16)
            idx_d[sl] = idx_d[sl] + N_NODES
            return c

        lax.fori_loop(0, _EP // 16, off_row, 0)

        def g1(ci):
            s = ci % 2
            return pltpu.async_copy(
                table_hbm.at[idx_s.at[pl.ds(ci * _C, _C)]], bufs[s], gsems[s])

        def g2(ci):
            s = ci % 2
            return pltpu.async_copy(
                table_hbm.at[idx_d.at[pl.ds(ci * _C, _C)]], bufs[s], gsems[s],
                add=True)

        def wb(ci):
            s = ci % 2
            return pltpu.async_copy(
                bufs[s], out_hbm.at[pl.ds(base + ci * _C, _C)], wsems[s])

        wbd = [None] * _NCHUNK
        d = g1(0)
        for ci in range(_NCHUNK):
            d.wait()
            dg2 = g2(ci)
            if ci >= 1:
                wbd[ci - 1].wait()
            if ci + 1 < _NCHUNK:
                d = g1(ci + 1)
            dg2.wait()
            wbd[ci] = wb(ci)
        wbd[_NCHUNK - 1].wait()

    _GATHER_ADD_CACHE[key] = gather_add
    return gather_add


_E_BLK = 6400


def _tail_body(g_ref, eft_ref, w1e_ref, b1_ref, w2_ref, b2t_ref, out_ref):
    x = (g_ref[...]
         + lax.dot_general(eft_ref[...], w1e_ref[...],
                           (((0,), (0,)), ((), ())),
                           preferred_element_type=jnp.float32)
         + b1_ref[...])
    h = x * jax.nn.sigmoid(x)
    out_ref[...] = (lax.dot_general(w2_ref[...], h,
                                    (((0,), (1,)), ((), ())),
                                    preferred_element_type=jnp.float32)
                    + b2t_ref[...])


def _tail_body_aliased(buf_ref, g_ref, eft_ref, w1e_ref, b1_ref, w2_ref,
                       b2t_ref, out_ref):
    del buf_ref
    _tail_body(g_ref, eft_ref, w1e_ref, b1_ref, w2_ref, b2t_ref, out_ref)


def _mlp_tail(g, ef_t, w1e, b1, w2, b2t, buf, ebase):
    n = g.shape[0]
    nblk = n // _E_BLK
    blk0 = ebase // _E_BLK
    col_spec = pl.BlockSpec((D_EDGE, _E_BLK), lambda i: (0, blk0 + i))
    data_specs = [
        pl.BlockSpec((_E_BLK, D_HIDDEN), lambda i: (i, 0)),
        col_spec,
        pl.BlockSpec((D_EDGE, D_HIDDEN), lambda i: (0, 0)),
        pl.BlockSpec((1, D_HIDDEN), lambda i: (0, 0)),
        pl.BlockSpec((D_HIDDEN, D_EDGE), lambda i: (0, 0)),
        pl.BlockSpec((D_EDGE, 1), lambda i: (0, 0)),
    ]
    out_shape = jax.ShapeDtypeStruct((D_EDGE, N_EDGES), jnp.float32)
    if buf is None:
        return pl.pallas_call(
            _tail_body,
            grid=(nblk,),
            in_specs=data_specs,
            out_specs=col_spec,
            out_shape=out_shape,
        )(g, ef_t, w1e, b1, w2, b2t)
    return pl.pallas_call(
        _tail_body_aliased,
        grid=(nblk,),
        in_specs=[pl.BlockSpec(memory_space=pl.ANY)] + data_specs,
        out_specs=col_spec,
        out_shape=out_shape,
        input_output_aliases={0: 0},
    )(buf, g, ef_t, w1e, b1, w2, b2t)


def kernel(node_scalars, edge_index, edge_feats, W1, b1, W2, b2):
    ei = edge_index.astype(jnp.int32)
    src = ei[0]
    dst = ei[1]
    ef_t = edge_feats.T
    w1e = W1[2 * D_SCALAR:]
    b1r = b1.reshape(1, D_HIDDEN)
    b2t = b2.reshape(D_EDGE, 1)
    table = _node_tables(node_scalars, W1)
    buf = None
    ebase = 0
    for sz in _SPLITS:
        g_k = _gather_add_build(ebase, sz)(table, src, dst)
        buf = _mlp_tail(g_k, ef_t, w1e, b1r, W2, b2t, buf, ebase)
        ebase += sz
    return buf.T
